# baseline (speedup 1.0000x reference)
import functools
import numpy as np
import jax, jax.numpy as jnp
from jax import lax
from jax.experimental import pallas as pl
from jax.experimental.pallas import tpu as pltpu
from jax.experimental.pallas import tpu_sc as plsc

S, D, H, E, K = 2048, 1024, 2048, 8, 2
BLK = 256
NB = (S * K) // BLK + E
NP = NB * BLK


def _gating_body(x_ref, wg_ref, bg_ref, logits_ref, idx_ref, w_ref):
    x = x_ref[...]
    lg = jax.lax.dot_general(x, wg_ref[...], (((1,), (0,)), ((), ())),
                             preferred_element_type=jnp.float32,
                             precision=jax.lax.Precision.DEFAULT)
    lg = lg + bg_ref[...]
    logits_ref[...] = lg
    ii = jax.lax.broadcasted_iota(jnp.int32, lg.shape, 1)
    m0 = jnp.max(lg, axis=1, keepdims=True)
    i0 = jnp.min(jnp.where(lg == m0, ii, E), axis=1, keepdims=True)
    lg2 = jnp.where(ii == i0, -jnp.inf, lg)
    m1 = jnp.max(lg2, axis=1, keepdims=True)
    i1 = jnp.min(jnp.where(lg2 == m1, ii, E), axis=1, keepdims=True)
    t = jnp.exp(m1 - m0)
    w0 = 1.0 / (1.0 + t)
    w1 = t / (1.0 + t)
    idx_ref[...] = jnp.concatenate([i0, i1], axis=1)
    w_ref[...] = jnp.concatenate([w0, w1], axis=1)


def _gating(x_flat, Wg, bg):
    return pl.pallas_call(
        _gating_body,
        out_shape=(
            jax.ShapeDtypeStruct((S, E), jnp.float32),
            jax.ShapeDtypeStruct((S, K), jnp.int32),
            jax.ShapeDtypeStruct((S, K), jnp.float32),
        ),
    )(x_flat, Wg, bg.reshape(1, E))


def _ffn_body(be_ref, nact_ref, xs_ref, w1_ref, b1_ref, w2_ref, b2_ref,
              out_ref):
    @pl.when(pl.program_id(0) < nact_ref[0])
    def _():
        h = jax.lax.dot_general(xs_ref[...], w1_ref[0], (((1,), (0,)), ((), ())),
                                preferred_element_type=jnp.float32,
                                precision=jax.lax.Precision.DEFAULT)
        h = h + b1_ref[0]
        h = 0.5 * h * (1.0 + jax.lax.erf(h / np.sqrt(2).astype(np.float32)))
        o = jax.lax.dot_general(h, w2_ref[0], (((1,), (0,)), ((), ())),
                                preferred_element_type=jnp.float32,
                                precision=jax.lax.Precision.DEFAULT)
        out_ref[...] = o + b2_ref[0]


def _ffn(xs, W1, b1, W2, b2, blk_expert, nact):
    grid_spec = pltpu.PrefetchScalarGridSpec(
        num_scalar_prefetch=2,
        grid=(NB,),
        in_specs=[
            pl.BlockSpec((BLK, D), lambda i, be, na: (i, 0)),
            pl.BlockSpec((1, D, H), lambda i, be, na: (be[i], 0, 0)),
            pl.BlockSpec((1, 1, H), lambda i, be, na: (be[i], 0, 0)),
            pl.BlockSpec((1, H, D), lambda i, be, na: (be[i], 0, 0)),
            pl.BlockSpec((1, 1, D), lambda i, be, na: (be[i], 0, 0)),
        ],
        out_specs=pl.BlockSpec((BLK, D), lambda i, be, na: (i, 0)),
    )
    return pl.pallas_call(
        _ffn_body,
        grid_spec=grid_spec,
        out_shape=jax.ShapeDtypeStruct((NP, D), jnp.float32),
        compiler_params=pltpu.CompilerParams(
            dimension_semantics=("arbitrary",)),
    )(blk_expert, nact, xs, W1, b1.reshape(E, 1, H), W2, b2.reshape(E, 1, D))


NW = 32
GCHUNK = 32


def _sc_gather(table, idx, n_rows):
    per_w = n_rows // NW
    nchunk = per_w // GCHUNK
    mesh = plsc.VectorSubcoreMesh(core_axis_name="c", subcore_axis_name="s")

    @functools.partial(
        pl.kernel, mesh=mesh,
        out_type=jax.ShapeDtypeStruct((n_rows, D), jnp.float32),
        scratch_types=[
            pltpu.VMEM((GCHUNK,), jnp.int32),
            pltpu.VMEM((GCHUNK,), jnp.int32),
            pltpu.VMEM((GCHUNK, D), jnp.float32),
            pltpu.VMEM((GCHUNK, D), jnp.float32),
            pltpu.SemaphoreType.DMA,
            pltpu.SemaphoreType.DMA,
        ],
    )
    def k(tab_hbm, idx_hbm, out_hbm, i0_v, i1_v, r0_v, r1_v, sem0, sem1):
        wid = lax.axis_index("s") * 2 + lax.axis_index("c")
        ivs, rvs, sems = (i0_v, i1_v), (r0_v, r1_v), (sem0, sem1)

        def base(c):
            return wid * per_w + c * GCHUNK

        pltpu.sync_copy(idx_hbm.at[pl.ds(base(0), GCHUNK)], i0_v)
        cps = {0: pltpu.async_copy(tab_hbm.at[i0_v], r0_v, sem0)}
        for c in range(nchunk):
            b, nb = c % 2, (c + 1) % 2
            if c + 1 < nchunk:
                pltpu.sync_copy(idx_hbm.at[pl.ds(base(c + 1), GCHUNK)], ivs[nb])
            cps[c].wait()
            if c + 1 < nchunk:
                cps[c + 1] = pltpu.async_copy(tab_hbm.at[ivs[nb]], rvs[nb],
                                              sems[nb])
            pltpu.sync_copy(rvs[b], out_hbm.at[pl.ds(base(c), GCHUNK)])

    return k(table, idx)


def _add_body(z_ref, w_ref, y_ref):
    wv = w_ref[...]
    y_ref[...] = (z_ref[:, 0, :] * wv[:, 0:1] + z_ref[:, 1, :] * wv[:, 1:2])


def _tc_add(z, w):
    return pl.pallas_call(
        _add_body,
        grid=(S // BLK,),
        in_specs=[
            pl.BlockSpec((BLK, 2, D), lambda i: (i, 0, 0)),
            pl.BlockSpec((BLK, 2), lambda i: (i, 0)),
        ],
        out_specs=pl.BlockSpec((BLK, D), lambda i: (i, 0)),
        out_shape=jax.ShapeDtypeStruct((S, D), jnp.float32),
    )(z, w)


def _route(idx):
    e = idx.reshape(-1)
    oh = jax.nn.one_hot(e, E, dtype=jnp.int32)
    counts = oh.sum(axis=0)
    rank = (jnp.cumsum(oh, axis=0) - oh)[jnp.arange(S * K), e]
    blocks_per_e = (counts + BLK - 1) // BLK
    blk_start_e = jnp.cumsum(blocks_per_e) - blocks_per_e
    pos = blk_start_e[e] * BLK + rank
    nact = jnp.sum(blocks_per_e)
    cumblocks = jnp.cumsum(blocks_per_e)
    bids = jnp.arange(NB, dtype=jnp.int32)
    blk_expert = jnp.minimum(
        jnp.searchsorted(cumblocks, bids, side="right").astype(jnp.int32), E - 1)
    sorted_token = (jnp.arange(NP, dtype=jnp.int32) % S).at[pos].set(
        jnp.arange(S * K, dtype=jnp.int32) // K)
    return pos, blk_expert, nact.reshape(1).astype(jnp.int32), sorted_token


def kernel(x, Wg, bg, W1, b1, W2, b2):
    Bx, Sx, Dx = x.shape
    x_flat = x.reshape(-1, Dx)
    logits, idx, w = _gating(x_flat, Wg, bg)
    pos, blk_expert, nact, sorted_token = _route(idx)
    xs = _sc_gather(x_flat, sorted_token, NP)
    ys = _ffn(xs, W1, b1, W2, b2, blk_expert, nact)
    z = _sc_gather(ys, pos, 2 * S)
    y = _tc_add(z.reshape(S, K, D), w)
    return (y.reshape(Bx, Sx, Dx), logits.reshape(Bx, Sx, E),
            idx.reshape(Bx, Sx, K))


# speedup vs baseline: 1.1076x; 1.1076x over previous
import functools
import numpy as np
import jax, jax.numpy as jnp
from jax import lax
from jax.experimental import pallas as pl
from jax.experimental.pallas import tpu as pltpu
from jax.experimental.pallas import tpu_sc as plsc

S, D, H, E, K = 2048, 1024, 2048, 8, 2
BLK = 256
NB = (S * K) // BLK + E
NP = NB * BLK


def _gating_body(x_ref, wg_ref, bg_ref, logits_ref, idx_ref, w_ref):
    x = x_ref[...]
    lg = jax.lax.dot_general(x, wg_ref[...], (((1,), (0,)), ((), ())),
                             preferred_element_type=jnp.float32,
                             precision=jax.lax.Precision.DEFAULT)
    lg = lg + bg_ref[...]
    logits_ref[...] = lg
    ii = jax.lax.broadcasted_iota(jnp.int32, lg.shape, 1)
    m0 = jnp.max(lg, axis=1, keepdims=True)
    i0 = jnp.min(jnp.where(lg == m0, ii, E), axis=1, keepdims=True)
    lg2 = jnp.where(ii == i0, -jnp.inf, lg)
    m1 = jnp.max(lg2, axis=1, keepdims=True)
    i1 = jnp.min(jnp.where(lg2 == m1, ii, E), axis=1, keepdims=True)
    t = jnp.exp(m1 - m0)
    w0 = 1.0 / (1.0 + t)
    w1 = t / (1.0 + t)
    idx_ref[...] = jnp.concatenate([i0, i1], axis=1)
    w_ref[...] = jnp.concatenate([w0, w1], axis=1)


def _gating(x_flat, Wg, bg):
    return pl.pallas_call(
        _gating_body,
        out_shape=(
            jax.ShapeDtypeStruct((S, E), jnp.float32),
            jax.ShapeDtypeStruct((S, K), jnp.int32),
            jax.ShapeDtypeStruct((S, K), jnp.float32),
        ),
    )(x_flat, Wg, bg.reshape(1, E))


def _ffn_body(be_ref, nact_ref, xs_ref, w1_ref, b1_ref, w2_ref, b2_ref,
              out_ref):
    @pl.when(pl.program_id(0) < nact_ref[0])
    def _():
        h = jax.lax.dot_general(xs_ref[...], w1_ref[0], (((1,), (0,)), ((), ())),
                                preferred_element_type=jnp.float32,
                                precision=jax.lax.Precision.DEFAULT)
        h = h + b1_ref[0]
        h = 0.5 * h * (1.0 + jax.lax.erf(h / np.sqrt(2).astype(np.float32)))
        o = jax.lax.dot_general(h, w2_ref[0], (((1,), (0,)), ((), ())),
                                preferred_element_type=jnp.float32,
                                precision=jax.lax.Precision.DEFAULT)
        out_ref[...] = o + b2_ref[0]


def _ffn(xs, W1, b1, W2, b2, blk_expert, nact):
    grid_spec = pltpu.PrefetchScalarGridSpec(
        num_scalar_prefetch=2,
        grid=(NB,),
        in_specs=[
            pl.BlockSpec((BLK, D), lambda i, be, na: (i, 0)),
            pl.BlockSpec((1, D, H), lambda i, be, na: (be[i], 0, 0)),
            pl.BlockSpec((1, 1, H), lambda i, be, na: (be[i], 0, 0)),
            pl.BlockSpec((1, H, D), lambda i, be, na: (be[i], 0, 0)),
            pl.BlockSpec((1, 1, D), lambda i, be, na: (be[i], 0, 0)),
        ],
        out_specs=pl.BlockSpec((BLK, D), lambda i, be, na: (i, 0)),
    )
    return pl.pallas_call(
        _ffn_body,
        grid_spec=grid_spec,
        out_shape=jax.ShapeDtypeStruct((NP, D), jnp.float32),
        compiler_params=pltpu.CompilerParams(
            dimension_semantics=("arbitrary",)),
    )(blk_expert, nact, xs, W1, b1.reshape(E, 1, H), W2, b2.reshape(E, 1, D))


NW = 32
GCHUNK = 32


def _sc_gather(table, idx, n_rows):
    per_w = n_rows // NW
    nchunk = per_w // GCHUNK
    mesh = plsc.VectorSubcoreMesh(core_axis_name="c", subcore_axis_name="s")

    @functools.partial(
        pl.kernel, mesh=mesh,
        out_type=jax.ShapeDtypeStruct((n_rows, D), jnp.float32),
        scratch_types=[
            pltpu.VMEM((GCHUNK,), jnp.int32),
            pltpu.VMEM((GCHUNK,), jnp.int32),
            pltpu.VMEM((GCHUNK, D), jnp.float32),
            pltpu.VMEM((GCHUNK, D), jnp.float32),
            pltpu.SemaphoreType.DMA,
            pltpu.SemaphoreType.DMA,
        ],
    )
    def k(tab_hbm, idx_hbm, out_hbm, i0_v, i1_v, r0_v, r1_v, sem0, sem1):
        wid = lax.axis_index("s") * 2 + lax.axis_index("c")
        ivs, rvs, sems = (i0_v, i1_v), (r0_v, r1_v), (sem0, sem1)

        def base(c):
            return wid * per_w + c * GCHUNK

        pltpu.sync_copy(idx_hbm.at[pl.ds(base(0), GCHUNK)], i0_v)
        cps = {0: pltpu.async_copy(tab_hbm.at[i0_v], r0_v, sem0)}
        for c in range(nchunk):
            b, nb = c % 2, (c + 1) % 2
            if c + 1 < nchunk:
                pltpu.sync_copy(idx_hbm.at[pl.ds(base(c + 1), GCHUNK)], ivs[nb])
            cps[c].wait()
            if c + 1 < nchunk:
                cps[c + 1] = pltpu.async_copy(tab_hbm.at[ivs[nb]], rvs[nb],
                                              sems[nb])
            pltpu.sync_copy(rvs[b], out_hbm.at[pl.ds(base(c), GCHUNK)])

    return k(table, idx)


def _add_body(a_ref, b_ref, w_ref, y_ref):
    wv = w_ref[...]
    y_ref[...] = a_ref[...] * wv[:, 0:1] + b_ref[...] * wv[:, 1:2]


def _tc_add(z, w):
    nb = S // BLK
    return pl.pallas_call(
        _add_body,
        grid=(nb,),
        in_specs=[
            pl.BlockSpec((BLK, D), lambda i: (i, 0)),
            pl.BlockSpec((BLK, D), lambda i: (nb + i, 0)),
            pl.BlockSpec((BLK, 2), lambda i: (i, 0)),
        ],
        out_specs=pl.BlockSpec((BLK, D), lambda i: (i, 0)),
        out_shape=jax.ShapeDtypeStruct((S, D), jnp.float32),
    )(z, z, w)


def _route(idx):
    e = idx.reshape(-1)
    oh = jax.nn.one_hot(e, E, dtype=jnp.int32)
    counts = oh.sum(axis=0)
    rank = (jnp.cumsum(oh, axis=0) - oh)[jnp.arange(S * K), e]
    blocks_per_e = (counts + BLK - 1) // BLK
    blk_start_e = jnp.cumsum(blocks_per_e) - blocks_per_e
    pos = blk_start_e[e] * BLK + rank
    nact = jnp.sum(blocks_per_e)
    cumblocks = jnp.cumsum(blocks_per_e)
    bids = jnp.arange(NB, dtype=jnp.int32)
    blk_expert = jnp.minimum(
        jnp.searchsorted(cumblocks, bids, side="right").astype(jnp.int32), E - 1)
    sorted_token = (jnp.arange(NP, dtype=jnp.int32) % S).at[pos].set(
        jnp.arange(S * K, dtype=jnp.int32) // K)
    return pos, blk_expert, nact.reshape(1).astype(jnp.int32), sorted_token


def kernel(x, Wg, bg, W1, b1, W2, b2):
    Bx, Sx, Dx = x.shape
    x_flat = x.reshape(-1, Dx)
    logits, idx, w = _gating(x_flat, Wg, bg)
    pos, blk_expert, nact, sorted_token = _route(idx)
    xs = _sc_gather(x_flat, sorted_token, NP)
    ys = _ffn(xs, W1, b1, W2, b2, blk_expert, nact)
    pos_sm = pos.reshape(S, K).T.reshape(-1)
    z = _sc_gather(ys, pos_sm, 2 * S)
    y = _tc_add(z, w)
    return (y.reshape(Bx, Sx, Dx), logits.reshape(Bx, Sx, E),
            idx.reshape(Bx, Sx, K))


# speedup vs baseline: 1.1213x; 1.0125x over previous
import functools
import numpy as np
import jax, jax.numpy as jnp
from jax import lax
from jax.experimental import pallas as pl
from jax.experimental.pallas import tpu as pltpu
from jax.experimental.pallas import tpu_sc as plsc

S, D, H, E, K = 2048, 1024, 2048, 8, 2
BLK = 512
CBLK = 256
NB = (S * K) // BLK + E
NP = NB * BLK


def _gating_body(x_ref, wg_ref, bg_ref, logits_ref, idx_ref, w_ref):
    x = x_ref[...]
    lg = jax.lax.dot_general(x, wg_ref[...], (((1,), (0,)), ((), ())),
                             preferred_element_type=jnp.float32,
                             precision=jax.lax.Precision.DEFAULT)
    lg = lg + bg_ref[...]
    logits_ref[...] = lg
    ii = jax.lax.broadcasted_iota(jnp.int32, lg.shape, 1)
    m0 = jnp.max(lg, axis=1, keepdims=True)
    i0 = jnp.min(jnp.where(lg == m0, ii, E), axis=1, keepdims=True)
    lg2 = jnp.where(ii == i0, -jnp.inf, lg)
    m1 = jnp.max(lg2, axis=1, keepdims=True)
    i1 = jnp.min(jnp.where(lg2 == m1, ii, E), axis=1, keepdims=True)
    t = jnp.exp(m1 - m0)
    w0 = 1.0 / (1.0 + t)
    w1 = t / (1.0 + t)
    idx_ref[...] = jnp.concatenate([i0, i1], axis=1)
    w_ref[...] = jnp.concatenate([w0, w1], axis=1)


def _gating(x_flat, Wg, bg):
    return pl.pallas_call(
        _gating_body,
        out_shape=(
            jax.ShapeDtypeStruct((S, E), jnp.float32),
            jax.ShapeDtypeStruct((S, K), jnp.int32),
            jax.ShapeDtypeStruct((S, K), jnp.float32),
        ),
    )(x_flat, Wg, bg.reshape(1, E))


def _ffn_body(be_ref, nact_ref, xs_ref, w1_ref, b1_ref, w2_ref, b2_ref,
              out_ref):
    @pl.when(pl.program_id(0) < nact_ref[0])
    def _():
        h = jax.lax.dot_general(xs_ref[...], w1_ref[0], (((1,), (0,)), ((), ())),
                                preferred_element_type=jnp.float32,
                                precision=jax.lax.Precision.DEFAULT)
        h = h + b1_ref[0]
        h = 0.5 * h * (1.0 + jax.lax.erf(h / np.sqrt(2).astype(np.float32)))
        o = jax.lax.dot_general(h, w2_ref[0], (((1,), (0,)), ((), ())),
                                preferred_element_type=jnp.float32,
                                precision=jax.lax.Precision.DEFAULT)
        out_ref[...] = o + b2_ref[0]


def _ffn(xs, W1, b1, W2, b2, blk_expert, nact):
    grid_spec = pltpu.PrefetchScalarGridSpec(
        num_scalar_prefetch=2,
        grid=(NB,),
        in_specs=[
            pl.BlockSpec((BLK, D), lambda i, be, na: (i, 0)),
            pl.BlockSpec((1, D, H), lambda i, be, na: (be[i], 0, 0)),
            pl.BlockSpec((1, 1, H), lambda i, be, na: (be[i], 0, 0)),
            pl.BlockSpec((1, H, D), lambda i, be, na: (be[i], 0, 0)),
            pl.BlockSpec((1, 1, D), lambda i, be, na: (be[i], 0, 0)),
        ],
        out_specs=pl.BlockSpec((BLK, D), lambda i, be, na: (i, 0)),
    )
    return pl.pallas_call(
        _ffn_body,
        grid_spec=grid_spec,
        out_shape=jax.ShapeDtypeStruct((NP, D), jnp.float32),
        compiler_params=pltpu.CompilerParams(
            dimension_semantics=("arbitrary",)),
    )(blk_expert, nact, xs, W1, b1.reshape(E, 1, H), W2, b2.reshape(E, 1, D))


NW = 32
GCHUNK = 32


def _sc_gather(table, idx, n_rows):
    per_w = n_rows // NW
    nchunk = per_w // GCHUNK
    mesh = plsc.VectorSubcoreMesh(core_axis_name="c", subcore_axis_name="s")

    @functools.partial(
        pl.kernel, mesh=mesh,
        out_type=jax.ShapeDtypeStruct((n_rows, D), jnp.float32),
        scratch_types=[
            pltpu.VMEM((GCHUNK,), jnp.int32),
            pltpu.VMEM((GCHUNK,), jnp.int32),
            pltpu.VMEM((GCHUNK, D), jnp.float32),
            pltpu.VMEM((GCHUNK, D), jnp.float32),
            pltpu.SemaphoreType.DMA,
            pltpu.SemaphoreType.DMA,
        ],
    )
    def k(tab_hbm, idx_hbm, out_hbm, i0_v, i1_v, r0_v, r1_v, sem0, sem1):
        wid = lax.axis_index("s") * 2 + lax.axis_index("c")
        ivs, rvs, sems = (i0_v, i1_v), (r0_v, r1_v), (sem0, sem1)

        def base(c):
            return wid * per_w + c * GCHUNK

        pltpu.sync_copy(idx_hbm.at[pl.ds(base(0), GCHUNK)], i0_v)
        cps = {0: pltpu.async_copy(tab_hbm.at[i0_v], r0_v, sem0)}
        for c in range(nchunk):
            b, nb = c % 2, (c + 1) % 2
            if c + 1 < nchunk:
                pltpu.sync_copy(idx_hbm.at[pl.ds(base(c + 1), GCHUNK)], ivs[nb])
            cps[c].wait()
            if c + 1 < nchunk:
                cps[c + 1] = pltpu.async_copy(tab_hbm.at[ivs[nb]], rvs[nb],
                                              sems[nb])
            pltpu.sync_copy(rvs[b], out_hbm.at[pl.ds(base(c), GCHUNK)])

    return k(table, idx)


def _add_body(a_ref, b_ref, w_ref, y_ref):
    wv = w_ref[...]
    y_ref[...] = a_ref[...] * wv[:, 0:1] + b_ref[...] * wv[:, 1:2]


def _tc_add(z, w):
    nb = S // CBLK
    return pl.pallas_call(
        _add_body,
        grid=(nb,),
        in_specs=[
            pl.BlockSpec((CBLK, D), lambda i: (i, 0)),
            pl.BlockSpec((CBLK, D), lambda i: (nb + i, 0)),
            pl.BlockSpec((CBLK, 2), lambda i: (i, 0)),
        ],
        out_specs=pl.BlockSpec((CBLK, D), lambda i: (i, 0)),
        out_shape=jax.ShapeDtypeStruct((S, D), jnp.float32),
    )(z, z, w)


def _route(idx):
    e = idx.reshape(-1)
    oh = jax.nn.one_hot(e, E, dtype=jnp.int32)
    counts = oh.sum(axis=0)
    rank = (jnp.cumsum(oh, axis=0) - oh)[jnp.arange(S * K), e]
    blocks_per_e = (counts + BLK - 1) // BLK
    blk_start_e = jnp.cumsum(blocks_per_e) - blocks_per_e
    pos = blk_start_e[e] * BLK + rank
    nact = jnp.sum(blocks_per_e)
    cumblocks = jnp.cumsum(blocks_per_e)
    bids = jnp.arange(NB, dtype=jnp.int32)
    blk_expert = jnp.minimum(
        jnp.searchsorted(cumblocks, bids, side="right").astype(jnp.int32), E - 1)
    sorted_token = (jnp.arange(NP, dtype=jnp.int32) % S).at[pos].set(
        jnp.arange(S * K, dtype=jnp.int32) // K)
    return pos, blk_expert, nact.reshape(1).astype(jnp.int32), sorted_token


def kernel(x, Wg, bg, W1, b1, W2, b2):
    Bx, Sx, Dx = x.shape
    x_flat = x.reshape(-1, Dx)
    logits, idx, w = _gating(x_flat, Wg, bg)
    pos, blk_expert, nact, sorted_token = _route(idx)
    xs = _sc_gather(x_flat, sorted_token, NP)
    ys = _ffn(xs, W1, b1, W2, b2, blk_expert, nact)
    pos_sm = pos.reshape(S, K).T.reshape(-1)
    z = _sc_gather(ys, pos_sm, 2 * S)
    y = _tc_add(z, w)
    return (y.reshape(Bx, Sx, Dx), logits.reshape(Bx, Sx, E),
            idx.reshape(Bx, Sx, K))


# speedup vs baseline: 1.4082x; 1.2558x over previous
import functools
import numpy as np
import jax, jax.numpy as jnp
from jax import lax
from jax.experimental import pallas as pl
from jax.experimental.pallas import tpu as pltpu
from jax.experimental.pallas import tpu_sc as plsc

S, D, H, E, K = 2048, 1024, 2048, 8, 2
BLK = 512
CBLK = 256
NB = (S * K) // BLK + E
NP = NB * BLK


def _gating_body(x_ref, wg_ref, bg_ref, logits_ref, idx_ref, w_ref):
    x = x_ref[...]
    lg = jax.lax.dot_general(x, wg_ref[...], (((1,), (0,)), ((), ())),
                             preferred_element_type=jnp.float32,
                             precision=jax.lax.Precision.DEFAULT)
    lg = lg + bg_ref[...]
    logits_ref[...] = lg
    ii = jax.lax.broadcasted_iota(jnp.int32, lg.shape, 1)
    m0 = jnp.max(lg, axis=1, keepdims=True)
    i0 = jnp.min(jnp.where(lg == m0, ii, E), axis=1, keepdims=True)
    lg2 = jnp.where(ii == i0, -jnp.inf, lg)
    m1 = jnp.max(lg2, axis=1, keepdims=True)
    i1 = jnp.min(jnp.where(lg2 == m1, ii, E), axis=1, keepdims=True)
    t = jnp.exp(m1 - m0)
    w0 = 1.0 / (1.0 + t)
    w1 = t / (1.0 + t)
    idx_ref[...] = jnp.concatenate([i0, i1], axis=1)
    w_ref[...] = jnp.concatenate([w0, w1], axis=1)


def _gating(x_flat, Wg, bg):
    return pl.pallas_call(
        _gating_body,
        out_shape=(
            jax.ShapeDtypeStruct((S, E), jnp.float32),
            jax.ShapeDtypeStruct((S, K), jnp.int32),
            jax.ShapeDtypeStruct((S, K), jnp.float32),
        ),
    )(x_flat, Wg, bg.reshape(1, E))


def _ffn_body(be_ref, nact_ref, xs_ref, w1_ref, b1_ref, w2_ref, b2_ref,
              out_ref):
    @pl.when(pl.program_id(0) < nact_ref[0])
    def _():
        h = jax.lax.dot_general(xs_ref[...], w1_ref[0], (((1,), (0,)), ((), ())),
                                preferred_element_type=jnp.float32,
                                precision=jax.lax.Precision.DEFAULT)
        h = h + b1_ref[0]
        h = 0.5 * h * (1.0 + jax.lax.erf(h / np.sqrt(2).astype(np.float32)))
        o = jax.lax.dot_general(h, w2_ref[0], (((1,), (0,)), ((), ())),
                                preferred_element_type=jnp.float32,
                                precision=jax.lax.Precision.DEFAULT)
        out_ref[...] = o + b2_ref[0]


def _ffn(xs, W1, b1, W2, b2, blk_expert, nact):
    grid_spec = pltpu.PrefetchScalarGridSpec(
        num_scalar_prefetch=2,
        grid=(NB,),
        in_specs=[
            pl.BlockSpec((BLK, D), lambda i, be, na: (i, 0)),
            pl.BlockSpec((1, D, H), lambda i, be, na: (be[i], 0, 0)),
            pl.BlockSpec((1, 1, H), lambda i, be, na: (be[i], 0, 0)),
            pl.BlockSpec((1, H, D), lambda i, be, na: (be[i], 0, 0)),
            pl.BlockSpec((1, 1, D), lambda i, be, na: (be[i], 0, 0)),
        ],
        out_specs=pl.BlockSpec((BLK, D), lambda i, be, na: (i, 0)),
    )
    return pl.pallas_call(
        _ffn_body,
        grid_spec=grid_spec,
        out_shape=jax.ShapeDtypeStruct((NP, D), jnp.float32),
        compiler_params=pltpu.CompilerParams(
            dimension_semantics=("arbitrary",)),
    )(blk_expert, nact, xs, W1, b1.reshape(E, 1, H), W2, b2.reshape(E, 1, D))


NW = 32
GCHUNK = 32


def _sc_gather(table, idx, n_rows):
    per_w = n_rows // NW
    nchunk = per_w // GCHUNK
    mesh = plsc.VectorSubcoreMesh(core_axis_name="c", subcore_axis_name="s")

    @functools.partial(
        pl.kernel, mesh=mesh,
        out_type=jax.ShapeDtypeStruct((n_rows, D), jnp.float32),
        scratch_types=[
            pltpu.VMEM((GCHUNK,), jnp.int32),
            pltpu.VMEM((GCHUNK,), jnp.int32),
            pltpu.VMEM((GCHUNK, D), jnp.float32),
            pltpu.VMEM((GCHUNK, D), jnp.float32),
            pltpu.SemaphoreType.DMA,
            pltpu.SemaphoreType.DMA,
        ],
    )
    def k(tab_hbm, idx_hbm, out_hbm, i0_v, i1_v, r0_v, r1_v, sem0, sem1):
        wid = lax.axis_index("s") * 2 + lax.axis_index("c")
        ivs, rvs, sems = (i0_v, i1_v), (r0_v, r1_v), (sem0, sem1)

        def base(c):
            return wid * per_w + c * GCHUNK

        pltpu.sync_copy(idx_hbm.at[pl.ds(base(0), GCHUNK)], i0_v)
        cps = {0: pltpu.async_copy(tab_hbm.at[i0_v], r0_v, sem0)}
        for c in range(nchunk):
            b, nb = c % 2, (c + 1) % 2
            if c + 1 < nchunk:
                pltpu.sync_copy(idx_hbm.at[pl.ds(base(c + 1), GCHUNK)], ivs[nb])
            cps[c].wait()
            if c + 1 < nchunk:
                cps[c + 1] = pltpu.async_copy(tab_hbm.at[ivs[nb]], rvs[nb],
                                              sems[nb])
            pltpu.sync_copy(rvs[b], out_hbm.at[pl.ds(base(c), GCHUNK)])

    return k(table, idx)


def _add_body(a_ref, b_ref, w_ref, y_ref):
    wv = w_ref[...]
    y_ref[...] = a_ref[...] * wv[:, 0:1] + b_ref[...] * wv[:, 1:2]


def _tc_add(z, w):
    nb = S // CBLK
    return pl.pallas_call(
        _add_body,
        grid=(nb,),
        in_specs=[
            pl.BlockSpec((CBLK, D), lambda i: (i, 0)),
            pl.BlockSpec((CBLK, D), lambda i: (nb + i, 0)),
            pl.BlockSpec((CBLK, 2), lambda i: (i, 0)),
        ],
        out_specs=pl.BlockSpec((CBLK, D), lambda i: (i, 0)),
        out_shape=jax.ShapeDtypeStruct((S, D), jnp.float32),
    )(z, z, w)


def _sc_route_scatter(x_flat, idx_sm):
    mesh = plsc.VectorSubcoreMesh(core_axis_name="c", subcore_axis_name="s")

    @functools.partial(
        pl.kernel, mesh=mesh,
        out_type=(
            jax.ShapeDtypeStruct((NP, D), jnp.float32),
            jax.ShapeDtypeStruct((S * K // 64, 64), jnp.int32),
            jax.ShapeDtypeStruct((32,), jnp.int32),
        ),
        scratch_types=[
            pltpu.VMEM((256,), jnp.int32),
            pltpu.VMEM((4, 64), jnp.int32),
            pltpu.VMEM((16,), jnp.int32),
            pltpu.VMEM((256,), jnp.int32),
            pltpu.VMEM((32,), jnp.int32),
            pltpu.VMEM((64, D), jnp.float32),
            pltpu.VMEM_SHARED((256,), jnp.int32),
            pltpu.SemaphoreType.DMA,
        ],
        compiler_params=pltpu.CompilerParams(needs_layout_passes=False),
    )
    def k(x_hbm, idxsm_hbm, xs_hbm, pos_hbm, meta_hbm,
          e_ref, pos_ref, cnt_ref, grid_ref, meta_ref, rows_v,
          shared, sem):
        wid = lax.axis_index("s")
        cid = lax.axis_index("c")
        lane = lax.iota(jnp.int32, 16)
        j0 = wid * 256
        pltpu.sync_copy(idxsm_hbm.at[pl.ds(j0, 256)], e_ref)

        counts_v = jnp.zeros((16,), jnp.int32)
        for r in range(16):
            ev = e_ref[pl.ds(r * 16, 16)]
            for e in range(E):
                c = plsc.all_reduce_population_count(ev == e)
                counts_v = jnp.where(lane == e, counts_v + c, counts_v)
        cnt_ref[...] = counts_v
        pltpu.sync_copy(cnt_ref, shared.at[pl.ds(wid * 16, 16)])
        plsc.subcore_barrier()

        pltpu.sync_copy(shared, grid_ref)
        tot = jnp.zeros((16,), jnp.int32)
        pre = jnp.zeros((16,), jnp.int32)
        for w2 in range(16):
            row = grid_ref[pl.ds(w2 * 16, 16)]
            tot = tot + row
            pre = jnp.where(w2 < wid, pre + row, pre)
        blocks_e = (tot + (BLK - 1)) >> 9
        cum_blocks = plsc.cumsum(blocks_e)
        blk_start_e = cum_blocks - blocks_e
        start_v = blk_start_e * BLK + pre

        @pl.when(jnp.logical_and(wid == 0, cid == 0))
        def _():
            acc = jnp.zeros((16,), jnp.int32)
            for e in range(E):
                ce = jnp.sum(jnp.where(lane == e, cum_blocks, 0))
                acc = acc + (lane >= ce).astype(jnp.int32)
            meta_ref[pl.ds(0, 16)] = jnp.minimum(acc, E - 1)
            nact = jnp.sum(blocks_e)
            meta_ref[pl.ds(16, 16)] = jnp.where(lane == 0, nact, 0)
            pltpu.sync_copy(meta_ref, meta_hbm)

        run_v = start_v
        for r in range(16):
            ev = e_ref[pl.ds(r * 16, 16)]
            pv = jnp.zeros((16,), jnp.int32)
            for e in range(E):
                m = ev == e
                mi = m.astype(jnp.int32)
                csum = plsc.cumsum(mi)
                se = jnp.sum(jnp.where(lane == e, run_v, 0))
                pv = jnp.where(m, se + csum - 1, pv)
                run_v = jnp.where(lane == e, run_v + jnp.sum(mi), run_v)
            pos_ref[r // 4, pl.ds((r % 4) * 16, 16)] = pv

        @pl.when(cid == 0)
        def _():
            pltpu.sync_copy(pos_ref, pos_hbm.at[pl.ds(wid * 4, 4)])

        t0 = (wid & 7) * 256
        for c2 in range(2):
            chunk = cid * 2 + c2
            pltpu.sync_copy(x_hbm.at[pl.ds(t0 + chunk * 64, 64)], rows_v)
            pltpu.async_copy(rows_v, xs_hbm.at[pos_ref.at[chunk]], sem).wait()

    return k(x_flat, idx_sm)


def kernel(x, Wg, bg, W1, b1, W2, b2):
    Bx, Sx, Dx = x.shape
    x_flat = x.reshape(-1, Dx)
    logits, idx, w = _gating(x_flat, Wg, bg)
    idx_sm = idx.T.reshape(-1)
    xs, pos2d, meta = _sc_route_scatter(x_flat, idx_sm)
    pos_sm = pos2d.reshape(-1)
    blk_expert, nact = meta[:NB], meta[NB:NB + 1]
    ys = _ffn(xs, W1, b1, W2, b2, blk_expert, nact)
    z = _sc_gather(ys, pos_sm, 2 * S)
    y = _tc_add(z, w)
    return (y.reshape(Bx, Sx, Dx), logits.reshape(Bx, Sx, E),
            idx.reshape(Bx, Sx, K))
